# Initial kernel scaffold; baseline (speedup 1.0000x reference)
import jax
import jax.numpy as jnp
from jax.experimental import pallas as pl


def kernel(features):
    raise NotImplementedError("write your pallas kernel here")


# speedup vs baseline: 4.5735x; 4.5735x over previous
import functools

import jax
import jax.numpy as jnp
import numpy as np
from jax.experimental import pallas as pl

_IRREPS = [(64, 1), (32, 3), (16, 5)]
_DIM = sum(m * d for m, d in _IRREPS)
_NSEG = sum(m for m, _ in _IRREPS)


def _selection_matrix() -> np.ndarray:
    s = np.zeros((_DIM, _NSEG), dtype=np.float32)
    col = 0
    seg = 0
    for mul, d in _IRREPS:
        for _ in range(mul):
            s[col:col + d, seg] = 1.0
            col += d
            seg += 1
    return s


def _norm_block(x_ref, s_ref, o_ref):
    x = x_ref[...]
    x2 = x * x
    sums = jnp.dot(x2, s_ref[...], preferred_element_type=jnp.float32)
    o_ref[...] = jnp.sqrt(sums)


def kernel(features):
    size = features.shape[:-1]
    x = features.reshape(-1, _DIM)
    n = x.shape[0]
    bn = 2000
    assert n % bn == 0
    sel = jnp.asarray(_selection_matrix())
    out = pl.pallas_call(
        _norm_block,
        grid=(n // bn,),
        in_specs=[
            pl.BlockSpec((bn, _DIM), lambda i: (i, 0)),
            pl.BlockSpec((_DIM, _NSEG), lambda i: (0, 0)),
        ],
        out_specs=pl.BlockSpec((bn, _NSEG), lambda i: (i, 0)),
        out_shape=jax.ShapeDtypeStruct((n, _NSEG), jnp.float32),
    )(x, sel)
    return out.reshape(size + (_NSEG,))
